# baseline (speedup 1.0000x reference)
import functools

import jax
import jax.numpy as jnp
from jax import lax
from jax.experimental import pallas as pl
from jax.experimental.pallas import tpu as pltpu
from jax.experimental.pallas import tpu_sc as plsc

S = 2048
D = 768
E = 8
H = 1536
O = 768
EXPAND = 2048
LOSS_COEF = 0.01

BLK = 256
NBLK = 23
NBUF = NBLK * BLK
NW = 32
CHUNK = S // NW


def _gate_body(x_ref, wg_ref, slot0_ref, slot1_ref, ws0_ref, ws1_ref,
               be_ref, loss_ref, excl_ref, mb_ref):
    x = x_ref[...]
    wg = wg_ref[...]
    logits = jnp.dot(x, wg, preferred_element_type=jnp.float32)

    lane_e = lax.broadcasted_iota(jnp.int32, (S, E), 1)
    neg = jnp.float32(-1e30)

    v1 = jnp.max(logits, axis=1, keepdims=True)
    i1 = jnp.min(jnp.where(logits == v1, lane_e, E), axis=1, keepdims=True)
    masked = jnp.where(lane_e == i1, neg, logits)
    v2 = jnp.max(masked, axis=1, keepdims=True)
    i2 = jnp.min(jnp.where(masked == v2, lane_e, E), axis=1, keepdims=True)

    t = jnp.exp(v2 - v1)
    g1 = 1.0 / (1.0 + t)
    g2 = t * g1

    m0 = (lane_e == i1).astype(jnp.float32)
    m1 = (lane_e == i2).astype(jnp.float32)
    mb = m0 + m1
    mb_ref[...] = mb

    rr = lax.broadcasted_iota(jnp.int32, (BLK, BLK), 0)
    cc = lax.broadcasted_iota(jnp.int32, (BLK, BLK), 1)
    tri = (cc < rr).astype(jnp.float32)

    def step(i, carry):
        off = pl.multiple_of(i * BLK, BLK)
        blk = mb_ref[pl.ds(off, BLK), :]
        excl_ref[pl.ds(off, BLK), :] = (
            jnp.dot(tri, blk, preferred_element_type=jnp.float32) + carry)
        return carry + jnp.sum(blk, axis=0, keepdims=True)

    counts_row = lax.fori_loop(0, S // BLK, step, jnp.zeros((1, E), jnp.float32))
    excl = excl_ref[...]

    padded_row = jnp.ceil(counts_row / BLK) * BLK
    re = lax.broadcasted_iota(jnp.int32, (E, E), 0)
    ce = lax.broadcasted_iota(jnp.int32, (E, E), 1)
    triu_strict = (re < ce).astype(jnp.float32)
    p_excl = jnp.dot(padded_row, triu_strict,
                     preferred_element_type=jnp.float32)

    base = excl + p_excl
    slot0 = jnp.sum(m0 * base, axis=1, keepdims=True)
    slot1 = jnp.sum(m1 * (base + m0), axis=1, keepdims=True)
    slot0_ref[...] = slot0.astype(jnp.int32)
    slot1_ref[...] = slot1.astype(jnp.int32)

    ws0_ref[...] = jnp.broadcast_to(g1, (S, 128))
    ws1_ref[...] = jnp.broadcast_to(g2, (S, 128))

    q_col = jnp.dot((ce <= re).astype(jnp.float32),
                    lax.dot_general(jnp.eye(E, dtype=jnp.float32), padded_row,
                                    (((1,), (1,)), ((), ()))),
                    preferred_element_type=jnp.float32)
    blk_id = (lax.broadcasted_iota(jnp.int32, (E, 128), 1) * BLK).astype(jnp.float32)
    ge = (blk_id >= q_col).astype(jnp.float32)
    be = jnp.dot(jnp.ones((1, E), jnp.float32), ge,
                 preferred_element_type=jnp.float32)
    emax = jnp.max(lax.broadcasted_iota(jnp.int32, (1, E), 1).astype(jnp.float32)
                   * (counts_row > 0).astype(jnp.float32))
    nb = jnp.sum(padded_row) * (1.0 / BLK)
    lane128 = lax.broadcasted_iota(jnp.int32, (1, 128), 1)
    be = jnp.where(lane128 == 127, nb, jnp.minimum(be, emax))
    be_ref[...] = be.astype(jnp.int32)

    c = jnp.sum((g1 > 0).astype(jnp.float32)) + jnp.sum((g2 > 0).astype(jnp.float32))
    n = jnp.float32(S * E)
    var = (c - c * c / n) / (n - 1.0)
    mean = c / n
    loss_ref[0, 0] = LOSS_COEF * var / (mean * mean + 1e-10)


def _gate_call(x2d, w_gate, *, interpret=False):
    out_shapes = (
        jax.ShapeDtypeStruct((S, 1), jnp.int32),
        jax.ShapeDtypeStruct((S, 1), jnp.int32),
        jax.ShapeDtypeStruct((S, 128), jnp.float32),
        jax.ShapeDtypeStruct((S, 128), jnp.float32),
        jax.ShapeDtypeStruct((1, 128), jnp.int32),
        jax.ShapeDtypeStruct((1, 1), jnp.float32),
    )
    return pl.pallas_call(
        _gate_body,
        out_shape=out_shapes,
        out_specs=(
            pl.BlockSpec(memory_space=pltpu.VMEM),
            pl.BlockSpec(memory_space=pltpu.VMEM),
            pl.BlockSpec(memory_space=pltpu.VMEM),
            pl.BlockSpec(memory_space=pltpu.VMEM),
            pl.BlockSpec(memory_space=pltpu.VMEM),
            pl.BlockSpec(memory_space=pltpu.SMEM),
        ),
        scratch_shapes=[pltpu.VMEM((S, E), jnp.float32),
                        pltpu.VMEM((S, E), jnp.float32)],
        interpret=interpret,
    )(x2d, w_gate)


def _dispatch_call(x2d, slot0, slot1, ws0, ws1, *, interpret=False):
    mesh = plsc.VectorSubcoreMesh(core_axis_name="c", subcore_axis_name="s",
                                  num_cores=2, num_subcores=16)

    @functools.partial(
        pl.kernel,
        out_type=(
            jax.ShapeDtypeStruct((NBUF, D), jnp.float32),
            jax.ShapeDtypeStruct((NBUF, 128), jnp.float32),
        ),
        mesh=mesh,
        scratch_types=[
            pltpu.VMEM((CHUNK,), jnp.int32),
            pltpu.VMEM((CHUNK,), jnp.int32),
            pltpu.VMEM((CHUNK, D), jnp.float32),
            pltpu.VMEM((CHUNK, 128), jnp.float32),
            pltpu.VMEM((CHUNK, 128), jnp.float32),
            pltpu.SemaphoreType.DMA,
        ],
        interpret=interpret,
    )
    def k(x_hbm, s0_hbm, s1_hbm, w0_hbm, w1_hbm, xs_hbm, ws_hbm,
          idx0_v, idx1_v, rows_v, g0_v, g1_v, sem):
        wid = lax.axis_index("s") * 2 + lax.axis_index("c")
        base = wid * CHUNK
        loads = [
            pltpu.async_copy(s0_hbm.at[pl.ds(base, CHUNK)], idx0_v, sem),
            pltpu.async_copy(s1_hbm.at[pl.ds(base, CHUNK)], idx1_v, sem),
            pltpu.async_copy(x_hbm.at[pl.ds(base, CHUNK)], rows_v, sem),
            pltpu.async_copy(w0_hbm.at[pl.ds(base, CHUNK)], g0_v, sem),
            pltpu.async_copy(w1_hbm.at[pl.ds(base, CHUNK)], g1_v, sem),
        ]
        for c in loads:
            c.wait()
        stores = [
            pltpu.async_copy(rows_v, xs_hbm.at[idx0_v], sem),
            pltpu.async_copy(rows_v, xs_hbm.at[idx1_v], sem),
            pltpu.async_copy(g0_v, ws_hbm.at[idx0_v], sem),
            pltpu.async_copy(g1_v, ws_hbm.at[idx1_v], sem),
        ]
        for c in stores:
            c.wait()

    return k(x2d, slot0, slot1, ws0, ws1)


KH = 2
HC = H // KH


def _mlp_body(be_ref, xs_ref, ws_ref, w1_ref, b1_ref, w2_ref, b2_ref, out_ref):
    b = pl.program_id(0)
    k = pl.program_id(1)
    nb = be_ref[127]

    @pl.when(b < nb)
    def _():
        xb = xs_ref[...]
        h = jnp.dot(xb, w1_ref[0], preferred_element_type=jnp.float32)
        h = jnp.maximum(h + b1_ref[0], 0.0)
        o = jnp.dot(h, w2_ref[0], preferred_element_type=jnp.float32)

        @pl.when(k == 0)
        def _():
            out_ref[...] = o

        @pl.when(k != 0)
        def _():
            out_ref[...] = out_ref[...] + o

        @pl.when(k == KH - 1)
        def _():
            out_ref[...] = (out_ref[...] + b2_ref[0]) * ws_ref[:, 0:1]


def _mlp_call(be_flat, xs, ws, fc1_w, fc1_b, fc2_w, fc2_b, *, interpret=False):
    grid_spec = pltpu.PrefetchScalarGridSpec(
        num_scalar_prefetch=1,
        grid=(NBLK, KH),
        in_specs=[
            pl.BlockSpec((BLK, D), lambda b, k, be: (b, 0)),
            pl.BlockSpec((BLK, 128), lambda b, k, be: (b, 0)),
            pl.BlockSpec((1, D, HC), lambda b, k, be: (be[b], 0, k)),
            pl.BlockSpec((1, 1, HC), lambda b, k, be: (be[b], 0, k)),
            pl.BlockSpec((1, HC, O), lambda b, k, be: (be[b], k, 0)),
            pl.BlockSpec((1, 1, O), lambda b, k, be: (be[b], 0, 0)),
        ],
        out_specs=pl.BlockSpec((BLK, O), lambda b, k, be: (b, 0)),
    )
    return pl.pallas_call(
        _mlp_body,
        grid_spec=grid_spec,
        out_shape=jax.ShapeDtypeStruct((NBUF, O), jnp.float32),
        interpret=interpret,
    )(be_flat, xs, ws, fc1_w, fc1_b.reshape(E, 1, H), fc2_w,
      fc2_b.reshape(E, 1, O))


def _combine_call(ys, slot0, slot1, *, interpret=False):
    mesh = plsc.VectorSubcoreMesh(core_axis_name="c", subcore_axis_name="s",
                                  num_cores=2, num_subcores=16)

    @functools.partial(
        pl.kernel,
        out_type=jax.ShapeDtypeStruct((S, O), jnp.float32),
        mesh=mesh,
        scratch_types=[
            pltpu.VMEM((CHUNK,), jnp.int32),
            pltpu.VMEM((CHUNK,), jnp.int32),
            pltpu.VMEM((CHUNK, O), jnp.float32),
            pltpu.VMEM((CHUNK, O), jnp.float32),
            pltpu.SemaphoreType.DMA,
        ],
        interpret=interpret,
    )
    def k(ys_hbm, s0_hbm, s1_hbm, y_hbm, idx0_v, idx1_v, v1, v2, sem):
        wid = lax.axis_index("s") * 2 + lax.axis_index("c")
        base = wid * CHUNK
        pltpu.sync_copy(s0_hbm.at[pl.ds(base, CHUNK)], idx0_v)
        pltpu.sync_copy(s1_hbm.at[pl.ds(base, CHUNK)], idx1_v)
        g1 = pltpu.async_copy(ys_hbm.at[idx0_v], v1, sem)
        g2 = pltpu.async_copy(ys_hbm.at[idx1_v], v2, sem)
        g1.wait()
        g2.wait()

        def add_row(r, _):
            for c in range(O // 16):
                v1[r, pl.ds(c * 16, 16)] = (
                    v1[r, pl.ds(c * 16, 16)] + v2[r, pl.ds(c * 16, 16)])
            return 0

        lax.fori_loop(0, CHUNK, add_row, 0)
        pltpu.sync_copy(v1, y_hbm.at[pl.ds(base, CHUNK)])

    return k(ys, slot0, slot1)


def kernel(x, expand_size, w_gate, fc1_w, fc1_b, fc2_w, fc2_b):
    x2d = x.reshape(S, D)
    slot0, slot1, ws0, ws1, be, lossv = _gate_call(x2d, w_gate)
    s0 = slot0.reshape(S)
    s1 = slot1.reshape(S)
    xs, ws = _dispatch_call(x2d, s0, s1, ws0, ws1)
    ys = _mlp_call(be.reshape(128), xs, ws, fc1_w, fc1_b, fc2_w, fc2_b)
    y2d = _combine_call(ys, s0, s1)
    y = y2d.reshape(1, S, O)
    if EXPAND != S:
        y = jnp.pad(y, ((0, 0), (0, EXPAND - S), (0, 0)))
    loss = lossv.reshape(())
    return (y, loss)


# speedup vs baseline: 1.3928x; 1.3928x over previous
import functools

import jax
import jax.numpy as jnp
from jax import lax
from jax.experimental import pallas as pl
from jax.experimental.pallas import tpu as pltpu
from jax.experimental.pallas import tpu_sc as plsc

S = 2048
D = 768
E = 8
H = 1536
O = 768
EXPAND = 2048
LOSS_COEF = 0.01

BLK = 256
NBLK = 23
NBUF = NBLK * BLK
NW = 32
CHUNK = S // NW


def _gate_body(x_ref, wg_ref, slot0_ref, slot1_ref, ws0_ref, ws1_ref,
               be_ref, loss_ref, excl_ref, mb_ref):
    x = x_ref[...]
    wg = wg_ref[...]
    logits = jnp.dot(x, wg, preferred_element_type=jnp.float32)

    lane_e = lax.broadcasted_iota(jnp.int32, (S, E), 1)
    neg = jnp.float32(-1e30)

    v1 = jnp.max(logits, axis=1, keepdims=True)
    i1 = jnp.min(jnp.where(logits == v1, lane_e, E), axis=1, keepdims=True)
    masked = jnp.where(lane_e == i1, neg, logits)
    v2 = jnp.max(masked, axis=1, keepdims=True)
    i2 = jnp.min(jnp.where(masked == v2, lane_e, E), axis=1, keepdims=True)

    t = jnp.exp(v2 - v1)
    g1 = 1.0 / (1.0 + t)
    g2 = t * g1

    m0 = (lane_e == i1).astype(jnp.float32)
    m1 = (lane_e == i2).astype(jnp.float32)
    mb = m0 + m1
    mb_ref[...] = mb

    rr = lax.broadcasted_iota(jnp.int32, (BLK, BLK), 0)
    cc = lax.broadcasted_iota(jnp.int32, (BLK, BLK), 1)
    tri = (cc < rr).astype(jnp.float32)

    def step(i, carry):
        off = pl.multiple_of(i * BLK, BLK)
        blk = mb_ref[pl.ds(off, BLK), :]
        excl_ref[pl.ds(off, BLK), :] = (
            jnp.dot(tri, blk, preferred_element_type=jnp.float32) + carry)
        return carry + jnp.sum(blk, axis=0, keepdims=True)

    counts_row = lax.fori_loop(0, S // BLK, step, jnp.zeros((1, E), jnp.float32))
    excl = excl_ref[...]

    padded_row = jnp.ceil(counts_row / BLK) * BLK
    re = lax.broadcasted_iota(jnp.int32, (E, E), 0)
    ce = lax.broadcasted_iota(jnp.int32, (E, E), 1)
    triu_strict = (re < ce).astype(jnp.float32)
    p_excl = jnp.dot(padded_row, triu_strict,
                     preferred_element_type=jnp.float32)

    base = excl + p_excl
    slot0 = jnp.sum(m0 * base, axis=1, keepdims=True)
    slot1 = jnp.sum(m1 * (base + m0), axis=1, keepdims=True)
    slot0_ref[...] = slot0.astype(jnp.int32)
    slot1_ref[...] = slot1.astype(jnp.int32)

    ws0_ref[...] = jnp.broadcast_to(g1, (S, 128))
    ws1_ref[...] = jnp.broadcast_to(g2, (S, 128))

    padded_col = lax.dot_general(jnp.eye(E, dtype=jnp.float32), padded_row,
                                 (((1,), (1,)), ((), ())))
    q_col = jnp.dot((ce <= re).astype(jnp.float32), padded_col,
                    preferred_element_type=jnp.float32)
    counts_col_pos = (padded_col > 0).astype(jnp.float32)
    blk_id = (lax.broadcasted_iota(jnp.int32, (E, 128), 1) * BLK).astype(jnp.float32)
    ones_row = jnp.ones((1, E), jnp.float32)
    ge = (blk_id >= q_col).astype(jnp.float32)
    be = jnp.dot(ones_row, ge, preferred_element_type=jnp.float32)
    emax = jnp.max(lax.broadcasted_iota(jnp.int32, (1, E), 1).astype(jnp.float32)
                   * (counts_row > 0).astype(jnp.float32))
    nb = jnp.sum(padded_row) * (1.0 / BLK)
    lane128 = lax.broadcasted_iota(jnp.int32, (1, 128), 1)
    be0 = jnp.where(lane128 == 127, nb, jnp.minimum(be, emax))
    ge_used = ge * counts_col_pos
    r_row = jnp.dot(ones_row, ge_used, preferred_element_type=jnp.float32)
    rank_col = jnp.dot((ce < re).astype(jnp.float32), counts_col_pos,
                       preferred_element_type=jnp.float32)
    eqn = (rank_col == r_row + 1.0).astype(jnp.float32) * counts_col_pos
    has = jnp.dot(ones_row, eqn, preferred_element_type=jnp.float32)
    e_row8 = lax.broadcasted_iota(jnp.int32, (1, E), 1).astype(jnp.float32)
    nxt_raw = jnp.dot(e_row8, eqn, preferred_element_type=jnp.float32)
    nxt_row = jnp.where(has > 0, nxt_raw, be0.astype(jnp.float32))
    start_col = q_col - padded_col
    eqstart = (blk_id == start_col).astype(jnp.float32) * counts_col_pos
    first_row = jnp.dot(ones_row, eqstart, preferred_element_type=jnp.float32)
    be_ref[...] = jnp.concatenate(
        [be0, r_row, nxt_row, first_row], axis=0).astype(jnp.int32)

    c = jnp.sum((g1 > 0).astype(jnp.float32)) + jnp.sum((g2 > 0).astype(jnp.float32))
    n = jnp.float32(S * E)
    var = (c - c * c / n) / (n - 1.0)
    mean = c / n
    loss_ref[0, 0] = LOSS_COEF * var / (mean * mean + 1e-10)


def _gate_call(x2d, w_gate, *, interpret=False):
    out_shapes = (
        jax.ShapeDtypeStruct((S, 1), jnp.int32),
        jax.ShapeDtypeStruct((S, 1), jnp.int32),
        jax.ShapeDtypeStruct((S, 128), jnp.float32),
        jax.ShapeDtypeStruct((S, 128), jnp.float32),
        jax.ShapeDtypeStruct((4, 128), jnp.int32),
        jax.ShapeDtypeStruct((1, 1), jnp.float32),
    )
    return pl.pallas_call(
        _gate_body,
        out_shape=out_shapes,
        out_specs=(
            pl.BlockSpec(memory_space=pltpu.VMEM),
            pl.BlockSpec(memory_space=pltpu.VMEM),
            pl.BlockSpec(memory_space=pltpu.VMEM),
            pl.BlockSpec(memory_space=pltpu.VMEM),
            pl.BlockSpec(memory_space=pltpu.VMEM),
            pl.BlockSpec(memory_space=pltpu.SMEM),
        ),
        scratch_shapes=[pltpu.VMEM((S, E), jnp.float32),
                        pltpu.VMEM((S, E), jnp.float32)],
        interpret=interpret,
    )(x2d, w_gate)


def _dispatch_call(x2d, slot0, slot1, ws0, ws1, *, interpret=False):
    mesh = plsc.VectorSubcoreMesh(core_axis_name="c", subcore_axis_name="s",
                                  num_cores=2, num_subcores=16)

    @functools.partial(
        pl.kernel,
        out_type=(
            jax.ShapeDtypeStruct((NBUF, D), jnp.float32),
            jax.ShapeDtypeStruct((NBUF, 128), jnp.float32),
        ),
        mesh=mesh,
        scratch_types=[
            pltpu.VMEM((CHUNK,), jnp.int32),
            pltpu.VMEM((CHUNK,), jnp.int32),
            pltpu.VMEM((CHUNK, D), jnp.float32),
            pltpu.VMEM((CHUNK, 128), jnp.float32),
            pltpu.VMEM((CHUNK, 128), jnp.float32),
            pltpu.SemaphoreType.DMA,
        ],
        interpret=interpret,
    )
    def k(x_hbm, s0_hbm, s1_hbm, w0_hbm, w1_hbm, xs_hbm, ws_hbm,
          idx0_v, idx1_v, rows_v, g0_v, g1_v, sem):
        wid = lax.axis_index("s") * 2 + lax.axis_index("c")
        base = wid * CHUNK
        loads = [
            pltpu.async_copy(s0_hbm.at[pl.ds(base, CHUNK)], idx0_v, sem),
            pltpu.async_copy(s1_hbm.at[pl.ds(base, CHUNK)], idx1_v, sem),
            pltpu.async_copy(x_hbm.at[pl.ds(base, CHUNK)], rows_v, sem),
            pltpu.async_copy(w0_hbm.at[pl.ds(base, CHUNK)], g0_v, sem),
            pltpu.async_copy(w1_hbm.at[pl.ds(base, CHUNK)], g1_v, sem),
        ]
        for c in loads:
            c.wait()
        stores = [
            pltpu.async_copy(rows_v, xs_hbm.at[idx0_v], sem),
            pltpu.async_copy(rows_v, xs_hbm.at[idx1_v], sem),
            pltpu.async_copy(g0_v, ws_hbm.at[idx0_v], sem),
            pltpu.async_copy(g1_v, ws_hbm.at[idx1_v], sem),
        ]
        for c in stores:
            c.wait()

    return k(x2d, slot0, slot1, ws0, ws1)


def _mlp_body(be_ref, xs_ref, ws_ref, w1_hbm, b1_ref, w2_hbm, b2_ref, out_ref,
              w1buf, w2buf, sem1, sem2):
    b = pl.program_id(0)
    nb = be_ref[0, 127]
    e = be_ref[0, b]
    r = be_ref[1, b]
    nxt = be_ref[2, b]
    first = be_ref[3, b]
    slot = lax.rem(r, 2)
    nslot = lax.rem(r + 1, 2)

    def start_fetch(expert, sl):
        pltpu.make_async_copy(w1_hbm.at[expert], w1buf.at[sl], sem1.at[sl]).start()
        pltpu.make_async_copy(w2_hbm.at[expert], w2buf.at[sl], sem2.at[sl]).start()

    def wait_fetch(sl):
        pltpu.make_async_copy(w1_hbm.at[0], w1buf.at[sl], sem1.at[sl]).wait()
        pltpu.make_async_copy(w2_hbm.at[0], w2buf.at[sl], sem2.at[sl]).wait()

    @pl.when(b == 0)
    def _():
        start_fetch(e, slot)

        @pl.when(nxt != e)
        def _():
            start_fetch(nxt, nslot)

        wait_fetch(slot)

    @pl.when(jnp.logical_and(b > 0,
                             jnp.logical_and(first == 1, b < nb)))
    def _():
        wait_fetch(slot)

        @pl.when(nxt != e)
        def _():
            start_fetch(nxt, nslot)

    @pl.when(b < nb)
    def _():
        xb = xs_ref[...]
        h = jnp.dot(xb, w1buf[slot], preferred_element_type=jnp.float32)
        h = jnp.maximum(h + b1_ref[0], 0.0)
        o = jnp.dot(h, w2buf[slot], preferred_element_type=jnp.float32)
        o = o + b2_ref[0]
        out_ref[...] = o * ws_ref[:, 0:1]


def _mlp_call(be_flat, xs, ws, fc1_w, fc1_b, fc2_w, fc2_b, *, interpret=False):
    grid_spec = pltpu.PrefetchScalarGridSpec(
        num_scalar_prefetch=1,
        grid=(NBLK,),
        in_specs=[
            pl.BlockSpec((BLK, D), lambda b, be: (b, 0)),
            pl.BlockSpec((BLK, 128), lambda b, be: (b, 0)),
            pl.BlockSpec(memory_space=pltpu.MemorySpace.HBM),
            pl.BlockSpec((1, 1, H), lambda b, be: (be[0, b], 0, 0)),
            pl.BlockSpec(memory_space=pltpu.MemorySpace.HBM),
            pl.BlockSpec((1, 1, O), lambda b, be: (be[0, b], 0, 0)),
        ],
        out_specs=pl.BlockSpec((BLK, O), lambda b, be: (b, 0)),
        scratch_shapes=[
            pltpu.VMEM((2, D, H), jnp.float32),
            pltpu.VMEM((2, H, O), jnp.float32),
            pltpu.SemaphoreType.DMA((2,)),
            pltpu.SemaphoreType.DMA((2,)),
        ],
    )
    return pl.pallas_call(
        _mlp_body,
        grid_spec=grid_spec,
        out_shape=jax.ShapeDtypeStruct((NBUF, O), jnp.float32),
        interpret=interpret,
    )(be_flat, xs, ws, fc1_w, fc1_b.reshape(E, 1, H), fc2_w,
      fc2_b.reshape(E, 1, O))


def _combine_call(ys, slot0, slot1, *, interpret=False):
    mesh = plsc.VectorSubcoreMesh(core_axis_name="c", subcore_axis_name="s",
                                  num_cores=2, num_subcores=16)

    @functools.partial(
        pl.kernel,
        out_type=jax.ShapeDtypeStruct((S, O), jnp.float32),
        mesh=mesh,
        scratch_types=[
            pltpu.VMEM((CHUNK,), jnp.int32),
            pltpu.VMEM((CHUNK,), jnp.int32),
            pltpu.VMEM((CHUNK, O), jnp.float32),
            pltpu.VMEM((CHUNK, O), jnp.float32),
            pltpu.SemaphoreType.DMA,
        ],
        interpret=interpret,
    )
    def k(ys_hbm, s0_hbm, s1_hbm, y_hbm, idx0_v, idx1_v, v1, v2, sem):
        wid = lax.axis_index("s") * 2 + lax.axis_index("c")
        base = wid * CHUNK
        pltpu.sync_copy(s0_hbm.at[pl.ds(base, CHUNK)], idx0_v)
        pltpu.sync_copy(s1_hbm.at[pl.ds(base, CHUNK)], idx1_v)
        g1 = pltpu.async_copy(ys_hbm.at[idx0_v], v1, sem)
        g2 = pltpu.async_copy(ys_hbm.at[idx1_v], v2, sem)
        g1.wait()
        g2.wait()

        def add_row(r, _):
            for c in range(O // 16):
                v1[r, pl.ds(c * 16, 16)] = (
                    v1[r, pl.ds(c * 16, 16)] + v2[r, pl.ds(c * 16, 16)])
            return 0

        lax.fori_loop(0, CHUNK, add_row, 0)
        pltpu.sync_copy(v1, y_hbm.at[pl.ds(base, CHUNK)])

    return k(ys, slot0, slot1)


def kernel(x, expand_size, w_gate, fc1_w, fc1_b, fc2_w, fc2_b):
    x2d = x.reshape(S, D)
    slot0, slot1, ws0, ws1, be, lossv = _gate_call(x2d, w_gate)
    s0 = slot0.reshape(S)
    s1 = slot1.reshape(S)
    xs, ws = _dispatch_call(x2d, s0, s1, ws0, ws1)
    ys = _mlp_call(be, xs, ws, fc1_w, fc1_b, fc2_w, fc2_b)
    y2d = _combine_call(ys, s0, s1)
    y = y2d.reshape(1, S, O)
    if EXPAND != S:
        y = jnp.pad(y, ((0, 0), (0, EXPAND - S), (0, 0)))
    loss = lossv.reshape(())
    return (y, loss)


# speedup vs baseline: 1.4185x; 1.0184x over previous
import functools

import jax
import jax.numpy as jnp
from jax import lax
from jax.experimental import pallas as pl
from jax.experimental.pallas import tpu as pltpu
from jax.experimental.pallas import tpu_sc as plsc

S = 2048
D = 768
E = 8
H = 1536
O = 768
EXPAND = 2048
LOSS_COEF = 0.01

BLK = 256
NBLK = 23
NBUF = NBLK * BLK
NW = 32
CHUNK = S // NW
WSW = 128


def _gate_body(x_ref, wg_ref, slot0_ref, slot1_ref, ws0_ref, ws1_ref,
               be_ref, loss_ref, excl_ref, mb_ref):
    x = x_ref[...]
    wg = wg_ref[...]
    logits = jnp.dot(x, wg, preferred_element_type=jnp.float32)

    lane_e = lax.broadcasted_iota(jnp.int32, (S, E), 1)
    neg = jnp.float32(-1e30)

    v1 = jnp.max(logits, axis=1, keepdims=True)
    i1 = jnp.min(jnp.where(logits == v1, lane_e, E), axis=1, keepdims=True)
    masked = jnp.where(lane_e == i1, neg, logits)
    v2 = jnp.max(masked, axis=1, keepdims=True)
    i2 = jnp.min(jnp.where(masked == v2, lane_e, E), axis=1, keepdims=True)

    t = jnp.exp(v2 - v1)
    g1 = 1.0 / (1.0 + t)
    g2 = t * g1

    m0 = (lane_e == i1).astype(jnp.float32)
    m1 = (lane_e == i2).astype(jnp.float32)
    mb = m0 + m1
    mb_ref[...] = mb

    rr = lax.broadcasted_iota(jnp.int32, (BLK, BLK), 0)
    cc = lax.broadcasted_iota(jnp.int32, (BLK, BLK), 1)
    tri = (cc < rr).astype(jnp.float32)

    def step(i, carry):
        off = pl.multiple_of(i * BLK, BLK)
        blk = mb_ref[pl.ds(off, BLK), :]
        excl_ref[pl.ds(off, BLK), :] = (
            jnp.dot(tri, blk, preferred_element_type=jnp.float32) + carry)
        return carry + jnp.sum(blk, axis=0, keepdims=True)

    counts_row = lax.fori_loop(0, S // BLK, step, jnp.zeros((1, E), jnp.float32))
    excl = excl_ref[...]

    padded_row = jnp.ceil(counts_row / BLK) * BLK
    re = lax.broadcasted_iota(jnp.int32, (E, E), 0)
    ce = lax.broadcasted_iota(jnp.int32, (E, E), 1)
    triu_strict = (re < ce).astype(jnp.float32)
    p_excl = jnp.dot(padded_row, triu_strict,
                     preferred_element_type=jnp.float32)

    base = excl + p_excl
    slot0 = jnp.sum(m0 * base, axis=1, keepdims=True)
    slot1 = jnp.sum(m1 * (base + m0), axis=1, keepdims=True)
    slot0_ref[...] = slot0.astype(jnp.int32)
    slot1_ref[...] = slot1.astype(jnp.int32)

    ws0_ref[...] = jnp.broadcast_to(g1, (S, WSW))
    ws1_ref[...] = jnp.broadcast_to(g2, (S, WSW))

    padded_col = lax.dot_general(jnp.eye(E, dtype=jnp.float32), padded_row,
                                 (((1,), (1,)), ((), ())))
    q_col = jnp.dot((ce <= re).astype(jnp.float32), padded_col,
                    preferred_element_type=jnp.float32)
    counts_col_pos = (padded_col > 0).astype(jnp.float32)
    blk_id = (lax.broadcasted_iota(jnp.int32, (E, 128), 1) * BLK).astype(jnp.float32)
    ones_row = jnp.ones((1, E), jnp.float32)
    ge = (blk_id >= q_col).astype(jnp.float32)
    be = jnp.dot(ones_row, ge, preferred_element_type=jnp.float32)
    emax = jnp.max(lax.broadcasted_iota(jnp.int32, (1, E), 1).astype(jnp.float32)
                   * (counts_row > 0).astype(jnp.float32))
    nb = jnp.sum(padded_row) * (1.0 / BLK)
    lane128 = lax.broadcasted_iota(jnp.int32, (1, 128), 1)
    be0 = jnp.where(lane128 == 127, nb, jnp.minimum(be, emax))
    ge_used = ge * counts_col_pos
    r_row = jnp.dot(ones_row, ge_used, preferred_element_type=jnp.float32)
    rank_col = jnp.dot((ce < re).astype(jnp.float32), counts_col_pos,
                       preferred_element_type=jnp.float32)
    eqn = (rank_col == r_row + 1.0).astype(jnp.float32) * counts_col_pos
    has = jnp.dot(ones_row, eqn, preferred_element_type=jnp.float32)
    e_row8 = lax.broadcasted_iota(jnp.int32, (1, E), 1).astype(jnp.float32)
    nxt_raw = jnp.dot(e_row8, eqn, preferred_element_type=jnp.float32)
    nxt_row = jnp.where(has > 0, nxt_raw, be0.astype(jnp.float32))
    start_col = q_col - padded_col
    eqstart = (blk_id == start_col).astype(jnp.float32) * counts_col_pos
    first_row = jnp.dot(ones_row, eqstart, preferred_element_type=jnp.float32)
    be_ref[...] = jnp.concatenate(
        [be0, r_row, nxt_row, first_row], axis=0).astype(jnp.int32)

    c = jnp.sum((g1 > 0).astype(jnp.float32)) + jnp.sum((g2 > 0).astype(jnp.float32))
    n = jnp.float32(S * E)
    var = (c - c * c / n) / (n - 1.0)
    mean = c / n
    loss_ref[0, 0] = LOSS_COEF * var / (mean * mean + 1e-10)


def _gate_call(x2d, w_gate, *, interpret=False):
    out_shapes = (
        jax.ShapeDtypeStruct((S, 1), jnp.int32),
        jax.ShapeDtypeStruct((S, 1), jnp.int32),
        jax.ShapeDtypeStruct((S, WSW), jnp.float32),
        jax.ShapeDtypeStruct((S, WSW), jnp.float32),
        jax.ShapeDtypeStruct((4, 128), jnp.int32),
        jax.ShapeDtypeStruct((1, 1), jnp.float32),
    )
    return pl.pallas_call(
        _gate_body,
        out_shape=out_shapes,
        out_specs=(
            pl.BlockSpec(memory_space=pltpu.VMEM),
            pl.BlockSpec(memory_space=pltpu.VMEM),
            pl.BlockSpec(memory_space=pltpu.VMEM),
            pl.BlockSpec(memory_space=pltpu.VMEM),
            pl.BlockSpec(memory_space=pltpu.VMEM),
            pl.BlockSpec(memory_space=pltpu.SMEM),
        ),
        scratch_shapes=[pltpu.VMEM((S, E), jnp.float32),
                        pltpu.VMEM((S, E), jnp.float32)],
        interpret=interpret,
    )(x2d, w_gate)


def _dispatch_call(x2d, slot0, slot1, ws0, ws1, *, interpret=False):
    mesh = plsc.VectorSubcoreMesh(core_axis_name="c", subcore_axis_name="s",
                                  num_cores=2, num_subcores=16)

    @functools.partial(
        pl.kernel,
        out_type=(
            jax.ShapeDtypeStruct((NBUF, D), jnp.float32),
            jax.ShapeDtypeStruct((NBUF, WSW), jnp.float32),
        ),
        mesh=mesh,
        scratch_types=[
            pltpu.VMEM((CHUNK,), jnp.int32),
            pltpu.VMEM((CHUNK,), jnp.int32),
            pltpu.VMEM((CHUNK, D), jnp.float32),
            pltpu.VMEM((CHUNK, WSW), jnp.float32),
            pltpu.VMEM((CHUNK, WSW), jnp.float32),
            pltpu.SemaphoreType.DMA,
        ],
        interpret=interpret,
    )
    def k(x_hbm, s0_hbm, s1_hbm, w0_hbm, w1_hbm, xs_hbm, ws_hbm,
          idx0_v, idx1_v, rows_v, g0_v, g1_v, sem):
        wid = lax.axis_index("s") * 2 + lax.axis_index("c")
        base = wid * CHUNK
        loads = [
            pltpu.async_copy(s0_hbm.at[pl.ds(base, CHUNK)], idx0_v, sem),
            pltpu.async_copy(s1_hbm.at[pl.ds(base, CHUNK)], idx1_v, sem),
            pltpu.async_copy(x_hbm.at[pl.ds(base, CHUNK)], rows_v, sem),
            pltpu.async_copy(w0_hbm.at[pl.ds(base, CHUNK)], g0_v, sem),
            pltpu.async_copy(w1_hbm.at[pl.ds(base, CHUNK)], g1_v, sem),
        ]
        for c in loads:
            c.wait()
        stores = [
            pltpu.async_copy(rows_v, xs_hbm.at[idx0_v], sem),
            pltpu.async_copy(rows_v, xs_hbm.at[idx1_v], sem),
            pltpu.async_copy(g0_v, ws_hbm.at[idx0_v], sem),
            pltpu.async_copy(g1_v, ws_hbm.at[idx1_v], sem),
        ]
        for c in stores:
            c.wait()

    return k(x2d, slot0, slot1, ws0, ws1)


def _mlp_body(be_ref, xs_ref, ws_ref, w1_hbm, b1_ref, w2_hbm, b2_ref, out_ref,
              w1buf, w2buf, sem1, sem2):
    b = pl.program_id(0)
    nb = be_ref[0, 127]
    e = be_ref[0, b]
    r = be_ref[1, b]
    nxt = be_ref[2, b]
    first = be_ref[3, b]
    slot = lax.rem(r, 2)
    nslot = lax.rem(r + 1, 2)

    def start_fetch(expert, sl):
        pltpu.make_async_copy(w1_hbm.at[expert], w1buf.at[sl], sem1.at[sl]).start()
        pltpu.make_async_copy(w2_hbm.at[expert], w2buf.at[sl], sem2.at[sl]).start()

    def wait_fetch(sl):
        pltpu.make_async_copy(w1_hbm.at[0], w1buf.at[sl], sem1.at[sl]).wait()
        pltpu.make_async_copy(w2_hbm.at[0], w2buf.at[sl], sem2.at[sl]).wait()

    @pl.when(b == 0)
    def _():
        start_fetch(e, slot)

        @pl.when(nxt != e)
        def _():
            start_fetch(nxt, nslot)

        wait_fetch(slot)

    @pl.when(jnp.logical_and(b > 0,
                             jnp.logical_and(first == 1, b < nb)))
    def _():
        wait_fetch(slot)

        @pl.when(nxt != e)
        def _():
            start_fetch(nxt, nslot)

    @pl.when(b < nb)
    def _():
        xb = xs_ref[...]
        h = jnp.dot(xb, w1buf[slot], preferred_element_type=jnp.float32)
        h = jnp.maximum(h + b1_ref[0], 0.0)
        o = jnp.dot(h, w2buf[slot], preferred_element_type=jnp.float32)
        o = o + b2_ref[0]
        out_ref[...] = o * ws_ref[:, 0:1]


def _mlp_call(be_flat, xs, ws, fc1_w, fc1_b, fc2_w, fc2_b, *, interpret=False):
    grid_spec = pltpu.PrefetchScalarGridSpec(
        num_scalar_prefetch=1,
        grid=(NBLK,),
        in_specs=[
            pl.BlockSpec((BLK, D), lambda b, be: (jnp.minimum(b, be[0, 127] - 1), 0)),
            pl.BlockSpec((BLK, WSW), lambda b, be: (jnp.minimum(b, be[0, 127] - 1), 0)),
            pl.BlockSpec(memory_space=pltpu.MemorySpace.HBM),
            pl.BlockSpec((1, 1, H), lambda b, be: (be[0, b], 0, 0)),
            pl.BlockSpec(memory_space=pltpu.MemorySpace.HBM),
            pl.BlockSpec((1, 1, O), lambda b, be: (be[0, b], 0, 0)),
        ],
        out_specs=pl.BlockSpec((BLK, O),
                               lambda b, be: (jnp.minimum(b, be[0, 127] - 1), 0)),
        scratch_shapes=[
            pltpu.VMEM((2, D, H), jnp.float32),
            pltpu.VMEM((2, H, O), jnp.float32),
            pltpu.SemaphoreType.DMA((2,)),
            pltpu.SemaphoreType.DMA((2,)),
        ],
    )
    return pl.pallas_call(
        _mlp_body,
        grid_spec=grid_spec,
        out_shape=jax.ShapeDtypeStruct((NBUF, O), jnp.float32),
        interpret=interpret,
    )(be_flat, xs, ws, fc1_w, fc1_b.reshape(E, 1, H), fc2_w,
      fc2_b.reshape(E, 1, O))


def _combine_call(ys, slot0, slot1, *, interpret=False):
    mesh = plsc.VectorSubcoreMesh(core_axis_name="c", subcore_axis_name="s",
                                  num_cores=2, num_subcores=16)

    @functools.partial(
        pl.kernel,
        out_type=jax.ShapeDtypeStruct((S, O), jnp.float32),
        mesh=mesh,
        scratch_types=[
            pltpu.VMEM((CHUNK,), jnp.int32),
            pltpu.VMEM((CHUNK,), jnp.int32),
            pltpu.VMEM((CHUNK, O), jnp.float32),
            pltpu.VMEM((CHUNK, O), jnp.float32),
            pltpu.SemaphoreType.DMA,
        ],
        interpret=interpret,
    )
    def k(ys_hbm, s0_hbm, s1_hbm, y_hbm, idx0_v, idx1_v, v1, v2, sem):
        wid = lax.axis_index("s") * 2 + lax.axis_index("c")
        base = wid * CHUNK
        pltpu.sync_copy(s0_hbm.at[pl.ds(base, CHUNK)], idx0_v)
        pltpu.sync_copy(s1_hbm.at[pl.ds(base, CHUNK)], idx1_v)
        g1 = pltpu.async_copy(ys_hbm.at[idx0_v], v1, sem)
        g2 = pltpu.async_copy(ys_hbm.at[idx1_v], v2, sem)
        g1.wait()
        g2.wait()

        def add_row(r, _):
            for c in range(O // 16):
                v1[r, pl.ds(c * 16, 16)] = (
                    v1[r, pl.ds(c * 16, 16)] + v2[r, pl.ds(c * 16, 16)])
            return 0

        lax.fori_loop(0, CHUNK, add_row, 0)
        pltpu.sync_copy(v1, y_hbm.at[pl.ds(base, CHUNK)])

    return k(ys, slot0, slot1)


def kernel(x, expand_size, w_gate, fc1_w, fc1_b, fc2_w, fc2_b):
    x2d = x.reshape(S, D)
    slot0, slot1, ws0, ws1, be, lossv = _gate_call(x2d, w_gate)
    s0 = slot0.reshape(S)
    s1 = slot1.reshape(S)
    xs, ws = _dispatch_call(x2d, s0, s1, ws0, ws1)
    ys = _mlp_call(be, xs, ws, fc1_w, fc1_b, fc2_w, fc2_b)
    y2d = _combine_call(ys, s0, s1)
    y = y2d.reshape(1, S, O)
    if EXPAND != S:
        y = jnp.pad(y, ((0, 0), (0, EXPAND - S), (0, 0)))
    loss = lossv.reshape(())
    return (y, loss)


# speedup vs baseline: 1.4356x; 1.0121x over previous
import functools

import jax
import jax.numpy as jnp
from jax import lax
from jax.experimental import pallas as pl
from jax.experimental.pallas import tpu as pltpu
from jax.experimental.pallas import tpu_sc as plsc

S = 2048
D = 768
E = 8
H = 1536
O = 768
EXPAND = 2048
LOSS_COEF = 0.01

BLK = 256
NBLK = 23
NBUF = NBLK * BLK
NW = 32
CHUNK = S // NW
WSW = 128


def _gate_body(x_ref, wg_ref, slot0_ref, slot1_ref, g1_ref, g2_ref,
               be_ref, loss_ref, excl_ref, mb_ref):
    x = x_ref[...]
    wg = wg_ref[...]
    logits = jnp.dot(x, wg, preferred_element_type=jnp.float32)

    lane_e = lax.broadcasted_iota(jnp.int32, (S, E), 1)
    neg = jnp.float32(-1e30)

    v1 = jnp.max(logits, axis=1, keepdims=True)
    i1 = jnp.min(jnp.where(logits == v1, lane_e, E), axis=1, keepdims=True)
    masked = jnp.where(lane_e == i1, neg, logits)
    v2 = jnp.max(masked, axis=1, keepdims=True)
    i2 = jnp.min(jnp.where(masked == v2, lane_e, E), axis=1, keepdims=True)

    t = jnp.exp(v2 - v1)
    g1 = 1.0 / (1.0 + t)
    g2 = t * g1

    m0 = (lane_e == i1).astype(jnp.float32)
    m1 = (lane_e == i2).astype(jnp.float32)
    mb = m0 + m1
    mb_ref[...] = mb

    rr = lax.broadcasted_iota(jnp.int32, (BLK, BLK), 0)
    cc = lax.broadcasted_iota(jnp.int32, (BLK, BLK), 1)
    tri = (cc < rr).astype(jnp.float32)

    def step(i, carry):
        off = pl.multiple_of(i * BLK, BLK)
        blk = mb_ref[pl.ds(off, BLK), :]
        excl_ref[pl.ds(off, BLK), :] = (
            jnp.dot(tri, blk, preferred_element_type=jnp.float32) + carry)
        return carry + jnp.sum(blk, axis=0, keepdims=True)

    counts_row = lax.fori_loop(0, S // BLK, step, jnp.zeros((1, E), jnp.float32))
    excl = excl_ref[...]

    padded_row = jnp.ceil(counts_row / BLK) * BLK
    re = lax.broadcasted_iota(jnp.int32, (E, E), 0)
    ce = lax.broadcasted_iota(jnp.int32, (E, E), 1)
    triu_strict = (re < ce).astype(jnp.float32)
    p_excl = jnp.dot(padded_row, triu_strict,
                     preferred_element_type=jnp.float32)

    base = excl + p_excl
    slot0 = jnp.sum(m0 * base, axis=1, keepdims=True)
    slot1 = jnp.sum(m1 * (base + m0), axis=1, keepdims=True)
    slot0_ref[...] = slot0.astype(jnp.int32)
    slot1_ref[...] = slot1.astype(jnp.int32)

    g1_ref[...] = jnp.broadcast_to(g1, (S, 16))
    g2_ref[...] = jnp.broadcast_to(g2, (S, 16))

    padded_col = lax.dot_general(jnp.eye(E, dtype=jnp.float32), padded_row,
                                 (((1,), (1,)), ((), ())))
    q_col = jnp.dot((ce <= re).astype(jnp.float32), padded_col,
                    preferred_element_type=jnp.float32)
    counts_col_pos = (padded_col > 0).astype(jnp.float32)
    blk_id = (lax.broadcasted_iota(jnp.int32, (E, 128), 1) * BLK).astype(jnp.float32)
    ones_row = jnp.ones((1, E), jnp.float32)
    ge = (blk_id >= q_col).astype(jnp.float32)
    be = jnp.dot(ones_row, ge, preferred_element_type=jnp.float32)
    emax = jnp.max(lax.broadcasted_iota(jnp.int32, (1, E), 1).astype(jnp.float32)
                   * (counts_row > 0).astype(jnp.float32))
    nb = jnp.sum(padded_row) * (1.0 / BLK)
    lane128 = lax.broadcasted_iota(jnp.int32, (1, 128), 1)
    be0 = jnp.where(lane128 == 127, nb, jnp.minimum(be, emax))
    ge_used = ge * counts_col_pos
    r_row = jnp.dot(ones_row, ge_used, preferred_element_type=jnp.float32)
    rank_col = jnp.dot((ce < re).astype(jnp.float32), counts_col_pos,
                       preferred_element_type=jnp.float32)
    eqn = (rank_col == r_row + 1.0).astype(jnp.float32) * counts_col_pos
    has = jnp.dot(ones_row, eqn, preferred_element_type=jnp.float32)
    e_row8 = lax.broadcasted_iota(jnp.int32, (1, E), 1).astype(jnp.float32)
    nxt_raw = jnp.dot(e_row8, eqn, preferred_element_type=jnp.float32)
    nxt_row = jnp.where(has > 0, nxt_raw, be0.astype(jnp.float32))
    start_col = q_col - padded_col
    eqstart = (blk_id == start_col).astype(jnp.float32) * counts_col_pos
    first_row = jnp.dot(ones_row, eqstart, preferred_element_type=jnp.float32)
    be_ref[...] = jnp.concatenate(
        [be0, r_row, nxt_row, first_row], axis=0).astype(jnp.int32)

    c = jnp.sum((g1 > 0).astype(jnp.float32)) + jnp.sum((g2 > 0).astype(jnp.float32))
    n = jnp.float32(S * E)
    var = (c - c * c / n) / (n - 1.0)
    mean = c / n
    loss_ref[0, 0] = LOSS_COEF * var / (mean * mean + 1e-10)


def _gate_call(x2d, w_gate, *, interpret=False):
    out_shapes = (
        jax.ShapeDtypeStruct((S, 1), jnp.int32),
        jax.ShapeDtypeStruct((S, 1), jnp.int32),
        jax.ShapeDtypeStruct((S, 16), jnp.float32),
        jax.ShapeDtypeStruct((S, 16), jnp.float32),
        jax.ShapeDtypeStruct((4, 128), jnp.int32),
        jax.ShapeDtypeStruct((1, 1), jnp.float32),
    )
    return pl.pallas_call(
        _gate_body,
        out_shape=out_shapes,
        out_specs=(
            pl.BlockSpec(memory_space=pltpu.VMEM),
            pl.BlockSpec(memory_space=pltpu.VMEM),
            pl.BlockSpec(memory_space=pltpu.VMEM),
            pl.BlockSpec(memory_space=pltpu.VMEM),
            pl.BlockSpec(memory_space=pltpu.VMEM),
            pl.BlockSpec(memory_space=pltpu.SMEM),
        ),
        scratch_shapes=[pltpu.VMEM((S, E), jnp.float32),
                        pltpu.VMEM((S, E), jnp.float32)],
        interpret=interpret,
    )(x2d, w_gate)


def _dispatch_call(x2d, slot0, slot1, *, interpret=False):
    mesh = plsc.VectorSubcoreMesh(core_axis_name="c", subcore_axis_name="s",
                                  num_cores=2, num_subcores=16)

    @functools.partial(
        pl.kernel,
        out_type=jax.ShapeDtypeStruct((NBUF, D), jnp.float32),
        mesh=mesh,
        scratch_types=[
            pltpu.VMEM((CHUNK,), jnp.int32),
            pltpu.VMEM((CHUNK,), jnp.int32),
            pltpu.VMEM((CHUNK, D), jnp.float32),
            pltpu.SemaphoreType.DMA,
        ],
        interpret=interpret,
    )
    def k(x_hbm, s0_hbm, s1_hbm, xs_hbm, idx0_v, idx1_v, rows_v, sem):
        wid = lax.axis_index("s") * 2 + lax.axis_index("c")
        base = wid * CHUNK
        loads = [
            pltpu.async_copy(s0_hbm.at[pl.ds(base, CHUNK)], idx0_v, sem),
            pltpu.async_copy(s1_hbm.at[pl.ds(base, CHUNK)], idx1_v, sem),
            pltpu.async_copy(x_hbm.at[pl.ds(base, CHUNK)], rows_v, sem),
        ]
        for c in loads:
            c.wait()
        stores = [
            pltpu.async_copy(rows_v, xs_hbm.at[idx0_v], sem),
            pltpu.async_copy(rows_v, xs_hbm.at[idx1_v], sem),
        ]
        for c in stores:
            c.wait()

    return k(x2d, slot0, slot1)


def _mlp_body(be_ref, xs_ref, w1_hbm, b1_ref, w2_hbm, b2_ref, out_ref,
              w1buf, w2buf, sem1, sem2):
    b = pl.program_id(0)
    nb = be_ref[0, 127]
    e = be_ref[0, b]
    r = be_ref[1, b]
    nxt = be_ref[2, b]
    first = be_ref[3, b]
    slot = lax.rem(r, 2)
    nslot = lax.rem(r + 1, 2)

    def start_fetch(expert, sl):
        pltpu.make_async_copy(w1_hbm.at[expert], w1buf.at[sl], sem1.at[sl]).start()
        pltpu.make_async_copy(w2_hbm.at[expert], w2buf.at[sl], sem2.at[sl]).start()

    def wait_fetch(sl):
        pltpu.make_async_copy(w1_hbm.at[0], w1buf.at[sl], sem1.at[sl]).wait()
        pltpu.make_async_copy(w2_hbm.at[0], w2buf.at[sl], sem2.at[sl]).wait()

    @pl.when(b == 0)
    def _():
        start_fetch(e, slot)

        @pl.when(nxt != e)
        def _():
            start_fetch(nxt, nslot)

        wait_fetch(slot)

    @pl.when(jnp.logical_and(b > 0,
                             jnp.logical_and(first == 1, b < nb)))
    def _():
        wait_fetch(slot)

        @pl.when(nxt != e)
        def _():
            start_fetch(nxt, nslot)

    @pl.when(b < nb)
    def _():
        xb = xs_ref[...]
        h = jnp.dot(xb, w1buf[slot], preferred_element_type=jnp.float32)
        h = jnp.maximum(h + b1_ref[0], 0.0)
        o = jnp.dot(h, w2buf[slot], preferred_element_type=jnp.float32)
        out_ref[...] = o + b2_ref[0]


def _mlp_call(be_flat, xs, fc1_w, fc1_b, fc2_w, fc2_b, *, interpret=False):
    grid_spec = pltpu.PrefetchScalarGridSpec(
        num_scalar_prefetch=1,
        grid=(NBLK,),
        in_specs=[
            pl.BlockSpec((BLK, D), lambda b, be: (jnp.minimum(b, be[0, 127] - 1), 0)),
            pl.BlockSpec(memory_space=pltpu.MemorySpace.HBM),
            pl.BlockSpec((1, 1, H), lambda b, be: (be[0, b], 0, 0)),
            pl.BlockSpec(memory_space=pltpu.MemorySpace.HBM),
            pl.BlockSpec((1, 1, O), lambda b, be: (be[0, b], 0, 0)),
        ],
        out_specs=pl.BlockSpec((BLK, O),
                               lambda b, be: (jnp.minimum(b, be[0, 127] - 1), 0)),
        scratch_shapes=[
            pltpu.VMEM((2, D, H), jnp.float32),
            pltpu.VMEM((2, H, O), jnp.float32),
            pltpu.SemaphoreType.DMA((2,)),
            pltpu.SemaphoreType.DMA((2,)),
        ],
    )
    return pl.pallas_call(
        _mlp_body,
        grid_spec=grid_spec,
        out_shape=jax.ShapeDtypeStruct((NBUF, O), jnp.float32),
        interpret=interpret,
    )(be_flat, xs, fc1_w, fc1_b.reshape(E, 1, H), fc2_w,
      fc2_b.reshape(E, 1, O))


def _combine_call(ys, slot0, slot1, g1c, g2c, *, interpret=False):
    mesh = plsc.VectorSubcoreMesh(core_axis_name="c", subcore_axis_name="s",
                                  num_cores=2, num_subcores=16)

    @functools.partial(
        pl.kernel,
        out_type=jax.ShapeDtypeStruct((S, O), jnp.float32),
        mesh=mesh,
        scratch_types=[
            pltpu.VMEM((CHUNK,), jnp.int32),
            pltpu.VMEM((CHUNK,), jnp.int32),
            pltpu.VMEM((CHUNK, 16), jnp.float32),
            pltpu.VMEM((CHUNK, 16), jnp.float32),
            pltpu.VMEM((CHUNK, O), jnp.float32),
            pltpu.VMEM((CHUNK, O), jnp.float32),
            pltpu.SemaphoreType.DMA,
        ],
        interpret=interpret,
    )
    def k(ys_hbm, s0_hbm, s1_hbm, g1_hbm, g2_hbm, y_hbm,
          idx0_v, idx1_v, gv1, gv2, v1, v2, sem):
        wid = lax.axis_index("s") * 2 + lax.axis_index("c")
        base = wid * CHUNK
        loads = [
            pltpu.async_copy(s0_hbm.at[pl.ds(base, CHUNK)], idx0_v, sem),
            pltpu.async_copy(s1_hbm.at[pl.ds(base, CHUNK)], idx1_v, sem),
            pltpu.async_copy(g1_hbm.at[pl.ds(base, CHUNK)], gv1, sem),
            pltpu.async_copy(g2_hbm.at[pl.ds(base, CHUNK)], gv2, sem),
        ]
        for c in loads:
            c.wait()
        d1 = pltpu.async_copy(ys_hbm.at[idx0_v], v1, sem)
        d2 = pltpu.async_copy(ys_hbm.at[idx1_v], v2, sem)
        d1.wait()
        d2.wait()

        def add_row(r, _):
            a = gv1[r, :]
            bgate = gv2[r, :]
            for c in range(O // 16):
                v1[r, pl.ds(c * 16, 16)] = (
                    v1[r, pl.ds(c * 16, 16)] * a
                    + v2[r, pl.ds(c * 16, 16)] * bgate)
            return 0

        lax.fori_loop(0, CHUNK, add_row, 0)
        pltpu.sync_copy(v1, y_hbm.at[pl.ds(base, CHUNK)])

    return k(ys, slot0, slot1, g1c, g2c)


def kernel(x, expand_size, w_gate, fc1_w, fc1_b, fc2_w, fc2_b):
    x2d = x.reshape(S, D)
    slot0, slot1, g1c, g2c, be, lossv = _gate_call(x2d, w_gate)
    s0 = slot0.reshape(S)
    s1 = slot1.reshape(S)
    xs = _dispatch_call(x2d, s0, s1)
    ys = _mlp_call(be, xs, fc1_w, fc1_b, fc2_w, fc2_b)
    y2d = _combine_call(ys, s0, s1, g1c, g2c)
    y = y2d.reshape(1, S, O)
    if EXPAND != S:
        y = jnp.pad(y, ((0, 0), (0, EXPAND - S), (0, 0)))
    loss = lossv.reshape(())
    return (y, loss)
